# baseline (speedup 1.0000x reference)
import functools

import jax
import jax.numpy as jnp
import numpy as np
from jax import lax
from jax.experimental import pallas as pl
from jax.experimental.pallas import tpu as pltpu
from jax.experimental.pallas import tpu_sc as plsc

N = 10000
E = 320000
P = 500000
D = 128
NRBF = 16
NC = 2
NS = 16
NW = NC * NS
L = 16

EPW = E // NW
ECH = 80
ENCH = EPW // ECH
ECHE = 40
ENCHE = EPW // ECHE
PCH = 80
RPT = 624
PCHT = P // PCH
NB = 10
NBLK = N // NB
EB = 4096
PB = 4096
F32 = jnp.float32


def _rsqrt(d2):
    ii = plsc.bitcast(d2, jnp.int32)
    y = plsc.bitcast(jnp.int32(0x5F3759DF) - (ii >> 1), F32)
    for _ in range(3):
        y = y * (1.5 - 0.5 * d2 * y * y)
    return y


def _dot(a, b):
    return jnp.dot(a, b, preferred_element_type=F32)


def _node_body(x_ref, w1_ref, b1_ref, w2_ref, b2_ref, wa_ref, wb_ref,
               bm_ref, h_ref, a_ref, b_ref):
    t = jnp.maximum(_dot(x_ref[...], w1_ref[...]) + b1_ref[...], 0.0)
    h = _dot(t, w2_ref[...]) + b2_ref[...]
    h_ref[...] = h
    a_ref[...] = _dot(h, wa_ref[...])
    b_ref[...] = _dot(h, wb_ref[...]) + bm_ref[...]


def _k_node(coords01, w1, b1, w2, b2, wa, wb, bm):
    full2 = pl.BlockSpec((D, D), lambda i: (0, 0))
    bias = pl.BlockSpec((1, D), lambda i: (0, 0))
    return pl.pallas_call(
        _node_body,
        grid=(NB,),
        in_specs=[
            pl.BlockSpec((NBLK, 2), lambda i: (i, 0)),
            pl.BlockSpec((2, D), lambda i: (0, 0)), bias, full2, bias,
            full2, full2, bias,
        ],
        out_specs=[pl.BlockSpec((NBLK, D), lambda i: (i, 0))] * 3,
        out_shape=[jax.ShapeDtypeStruct((N, D), F32)] * 3,
    )(coords01, w1, b1, w2, b2, wa, wb, bm)


def _make_rbfc_body(nw):
    def body(*refs):
        r_ref = refs[0]
        w_refs = refs[1:1 + nw]
        c_refs = refs[1 + nw:]
        centers = lax.broadcasted_iota(jnp.int32, (NRBF, 1), 0).astype(F32) \
            * np.float32(1.5 / (NRBF - 1))
        ft = jnp.exp(-10.0 * (r_ref[...][None, :] - centers) ** 2)
        dn = (((0,), (0,)), ((), ()))
        for w_ref, c_ref in zip(w_refs, c_refs):
            c_ref[...] = lax.dot_general(ft, w_ref[...], dn,
                                         preferred_element_type=F32)
    return body


def _k_rbfc(r, ws):
    wspec = pl.BlockSpec((NRBF, D), lambda i: (0, 0))
    return pl.pallas_call(
        _make_rbfc_body(len(ws)),
        grid=(pl.cdiv(E, EB),),
        in_specs=[pl.BlockSpec((EB,), lambda i: (i,))] + [wspec] * len(ws),
        out_specs=[pl.BlockSpec((EB, D), lambda i: (i, 0))] * len(ws),
        out_shape=[jax.ShapeDtypeStruct((E, D), F32)] * len(ws),
    )(r, *ws)


def _post_body(h_ref, aga_ref, agb_ref, dga_ref, dgb_ref, w2m_ref, b2m_ref,
               u1a_ref, u1b_ref, ub1_ref, u2_ref, ub2_ref, nwa_ref, nwb_ref,
               nb1_ref, hn_ref, a_ref, b_ref):
    h = h_ref[...]
    aggp = aga_ref[...] + agb_ref[...]
    deg = jnp.sum(dga_ref[...] + dgb_ref[...], axis=1, keepdims=True)
    agg = _dot(aggp, w2m_ref[...]) + deg * b2m_ref[...]
    t = jnp.maximum(_dot(h, u1a_ref[...]) + _dot(agg, u1b_ref[...])
                    + ub1_ref[...], 0.0)
    hn = h + _dot(t, u2_ref[...]) + ub2_ref[...]
    hn_ref[...] = hn
    a_ref[...] = _dot(hn, nwa_ref[...])
    b_ref[...] = _dot(hn, nwb_ref[...]) + nb1_ref[...]


def _k_post(h, agg2, deg2, w2m, b2m, u1a, u1b, ub1, u2, ub2, nwa, nwb, nb1):
    full2 = pl.BlockSpec((D, D), lambda i: (0, 0))
    bias = pl.BlockSpec((1, D), lambda i: (0, 0))
    nblk = pl.BlockSpec((NBLK, D), lambda i: (i, 0))
    return pl.pallas_call(
        _post_body,
        grid=(NB,),
        in_specs=[
            nblk,
            pl.BlockSpec((NBLK, D), lambda i: (i, 0)),
            pl.BlockSpec((NBLK, D), lambda i: (i + NB, 0)),
            pl.BlockSpec((NBLK, NRBF), lambda i: (i, 0)),
            pl.BlockSpec((NBLK, NRBF), lambda i: (i + NB, 0)),
            full2, bias, full2, full2, bias, full2, bias,
            full2, full2, bias,
        ],
        out_specs=[nblk] * 3,
        out_shape=[jax.ShapeDtypeStruct((N, D), F32)] * 3,
    )(h, agg2, agg2, deg2, deg2, w2m, b2m, u1a, u1b, ub1, u2, ub2,
      nwa, nwb, nb1)


def _geom_body(x_hbm, y_hbm, src_hbm, dst_hbm, ones_hbm, zdeg_hbm,
               r_hbm, deg_hbm,
               si0, si1, si2, si3, di0, di1, di2, di3,
               x_v, y_v, r_v, ones_v, deg_sp,
               ssi0, ssi1, ssi2, ssi3, sdi0, sdi1, sdi2, sdi3):
    si = [si0, si1, si2, si3]
    di = [di0, di1, di2, di3]
    ssi = [ssi0, ssi1, ssi2, ssi3]
    sdi = [sdi0, sdi1, sdi2, sdi3]
    cid = lax.axis_index("c")
    sid = lax.axis_index("s")
    wid = cid * NS + sid
    pltpu.sync_copy(x_hbm, x_v)
    pltpu.sync_copy(y_hbm, y_v)
    pltpu.sync_copy(ones_hbm, ones_v)

    @pl.when(sid == 0)
    def _():
        pltpu.sync_copy(zdeg_hbm, deg_sp)

    plsc.subcore_barrier()

    def fire_idx(k, q):
        base = wid * EPW + k * ECH
        pltpu.async_copy(src_hbm.at[pl.ds(base, ECH)], si[q], ssi[q])
        pltpu.async_copy(dst_hbm.at[pl.ds(base, ECH)], di[q], sdi[q])

    def wait_idx(q):
        pltpu.make_async_copy(src_hbm.at[pl.ds(0, ECH)], si[q], ssi[q]).wait()
        pltpu.make_async_copy(dst_hbm.at[pl.ds(0, ECH)], di[q], sdi[q]).wait()

    fire_idx(0, 0)
    fire_idx(1, 1)

    def quad(kk, carry):
        for j in range(4):
            k = 4 * kk + j

            @pl.when(k + 2 < ENCH)
            def _(k=k, q=(j + 2) % 4):
                fire_idx(k + 2, q)

            @pl.when(k < ENCH)
            def _(k=k, q=j):
                wait_idx(q)
                base = wid * EPW + k * ECH
                for jj in range(ECH // L):
                    sl = pl.ds(jj * L, L)
                    s16 = si[q][sl]
                    d16 = di[q][sl]
                    dx = (plsc.load_gather(x_v, [s16])
                          - plsc.load_gather(x_v, [d16]))
                    dy = (plsc.load_gather(y_v, [s16])
                          - plsc.load_gather(y_v, [d16]))
                    d2 = dx * dx + dy * dy + 1e-8
                    r_v[sl] = d2 * _rsqrt(d2)
                pltpu.sync_copy(r_v, r_hbm.at[pl.ds(base, ECH)])
                pltpu.sync_copy(ones_v, deg_sp.at[di[q]], add=True)
        return carry

    lax.fori_loop(0, (ENCH + 3) // 4, quad, 0)
    plsc.subcore_barrier()
    pltpu.sync_copy(deg_sp.at[pl.ds(sid * RPT, RPT)],
                    deg_hbm.at[pl.ds(cid * N + sid * RPT, RPT)])

    @pl.when(sid == 0)
    def _():
        pltpu.sync_copy(deg_sp.at[pl.ds(NS * RPT, N - NS * RPT)],
                        deg_hbm.at[pl.ds(cid * N + NS * RPT, N - NS * RPT)])


def _k_geom(*args):
    mesh = plsc.VectorSubcoreMesh(core_axis_name="c", subcore_axis_name="s")
    return pl.kernel(
        _geom_body,
        out_type=[jax.ShapeDtypeStruct((E,), F32),
                  jax.ShapeDtypeStruct((2 * N, NRBF), F32)],
        mesh=mesh,
        compiler_params=pltpu.CompilerParams(needs_layout_passes=False),
        scratch_types=(
            [pltpu.VMEM((ECH,), jnp.int32)] * 8
            + [pltpu.VMEM((N,), F32),
               pltpu.VMEM((N,), F32),
               pltpu.VMEM((ECH,), F32),
               pltpu.VMEM((ECH, NRBF), F32),
               pltpu.VMEM_SHARED((N, NRBF), F32)]
            + [pltpu.SemaphoreType.DMA] * 8
        ),
    )(*args)


def _edge_body(a_hbm, b_hbm, c_hbm, src_hbm, dst_hbm, zn_hbm, agg_hbm,
               si0, si1, si2, si3, di0, di1, di2, di3,
               a0, a1, a2, b0, b1, b2, c0, c1, c2, acc_sp,
               ssi0, ssi1, ssi2, ssi3, sdi0, sdi1, sdi2, sdi3,
               sa0, sa1, sa2, sb0, sb1, sb2, sc0, sc1, sc2,
               ss0, ss1, ss2):
    si = [si0, si1, si2, si3]
    di = [di0, di1, di2, di3]
    ab = [a0, a1, a2]
    bb = [b0, b1, b2]
    cb = [c0, c1, c2]
    ssi = [ssi0, ssi1, ssi2, ssi3]
    sdi = [sdi0, sdi1, sdi2, sdi3]
    sa = [sa0, sa1, sa2]
    sb = [sb0, sb1, sb2]
    sc = [sc0, sc1, sc2]
    ss = [ss0, ss1, ss2]
    cid = lax.axis_index("c")
    sid = lax.axis_index("s")
    wid = cid * NS + sid

    @pl.when(sid == 0)
    def _():
        pltpu.sync_copy(zn_hbm, acc_sp)

    plsc.subcore_barrier()

    def fire_idx(k, q):
        base = wid * EPW + k * ECHE
        pltpu.async_copy(src_hbm.at[pl.ds(base, ECHE)], si[q], ssi[q])
        pltpu.async_copy(dst_hbm.at[pl.ds(base, ECHE)], di[q], sdi[q])

    def wait_idx(q):
        pltpu.make_async_copy(src_hbm.at[pl.ds(0, ECHE)], si[q], ssi[q]).wait()
        pltpu.make_async_copy(dst_hbm.at[pl.ds(0, ECHE)], di[q], sdi[q]).wait()

    def fire_rows(k, q, s):
        base = wid * EPW + k * ECHE
        pltpu.async_copy(a_hbm.at[si[q]], ab[s], sa[s])
        pltpu.async_copy(b_hbm.at[di[q]], bb[s], sb[s])
        pltpu.async_copy(c_hbm.at[pl.ds(base, ECHE)], cb[s], sc[s])

    def wait_rows(s):
        pltpu.make_async_copy(a_hbm.at[pl.ds(0, ECHE)], ab[s], sa[s]).wait()
        pltpu.make_async_copy(b_hbm.at[pl.ds(0, ECHE)], bb[s], sb[s]).wait()
        pltpu.make_async_copy(c_hbm.at[pl.ds(0, ECHE)], cb[s], sc[s]).wait()

    def wait_scat(q, s):
        pltpu.make_async_copy(ab[s], acc_sp.at[di[q]], ss[s]).wait()

    fire_idx(0, 0)
    fire_idx(1, 1)
    wait_idx(0)
    fire_rows(0, 0, 0)

    def blk(kk, carry):
        for j in range(12):
            k = 12 * kk + j

            @pl.when(jnp.logical_and(k >= 2, k - 2 < ENCHE))
            def _(q=(j + 2) % 4, s=(j + 1) % 3):
                wait_scat(q, s)

            @pl.when(k + 2 < ENCHE)
            def _(k=k, q=(j + 2) % 4):
                fire_idx(k + 2, q)

            @pl.when(k + 1 < ENCHE)
            def _(k=k, q=(j + 1) % 4, s=(j + 1) % 3):
                wait_idx(q)
                fire_rows(k + 1, q, s)

            @pl.when(k < ENCHE)
            def _(k=k, q=j % 4, s=j % 3):
                wait_rows(s)

                @plsc.parallel_loop(0, ECHE, 1, unroll=4)
                def _(i):
                    for jj in range(D // L):
                        sl = pl.ds(jj * L, L)
                        z = ab[s][i, sl] + bb[s][i, sl] + cb[s][i, sl]
                        ab[s][i, sl] = jnp.maximum(z, 0.0)

                pltpu.async_copy(ab[s], acc_sp.at[di[q]], ss[s], add=True)
        return carry

    lax.fori_loop(0, (ENCHE + 11) // 12, blk, 0)
    plsc.subcore_barrier()
    pltpu.sync_copy(acc_sp.at[pl.ds(sid * RPT, RPT)],
                    agg_hbm.at[pl.ds(cid * N + sid * RPT, RPT)])

    @pl.when(sid == 0)
    def _():
        pltpu.sync_copy(acc_sp.at[pl.ds(NS * RPT, N - NS * RPT)],
                        agg_hbm.at[pl.ds(cid * N + NS * RPT, N - NS * RPT)])


def _k_edge(*args):
    mesh = plsc.VectorSubcoreMesh(core_axis_name="c", subcore_axis_name="s")
    return pl.kernel(
        _edge_body,
        out_type=jax.ShapeDtypeStruct((2 * N, D), F32),
        mesh=mesh,
        compiler_params=pltpu.CompilerParams(needs_layout_passes=False),
        scratch_types=(
            [pltpu.VMEM((ECHE,), jnp.int32)] * 8
            + [pltpu.VMEM((ECHE, D), F32)] * 9
            + [pltpu.VMEM_SHARED((N, D), F32)]
            + [pltpu.SemaphoreType.DMA] * 20
        ),
    )(*args)


def _head_body(hu_hbm, hv_hbm, x_hbm, y_hbm, u_hbm, v_hbm,
               g_hbm, r_hbm,
               ui0, ui1, ui2, ui3, vi0, vi1, vi2, vi3,
               a0, a1, a2, b0, b1, b2,
               x_v, y_v, r0, r1,
               sui0, sui1, sui2, sui3, svi0, svi1, svi2, svi3,
               sa0, sa1, sa2, sb0, sb1, sb2, sg0, sg1, sg2, sr0, sr1):
    ui = [ui0, ui1, ui2, ui3]
    vi = [vi0, vi1, vi2, vi3]
    ab = [a0, a1, a2]
    bb = [b0, b1, b2]
    rv = [r0, r1]
    sui = [sui0, sui1, sui2, sui3]
    svi = [svi0, svi1, svi2, svi3]
    sa = [sa0, sa1, sa2]
    sb = [sb0, sb1, sb2]
    sg = [sg0, sg1, sg2]
    sr = [sr0, sr1]
    cid = lax.axis_index("c")
    sid = lax.axis_index("s")
    wid = cid * NS + sid
    pltpu.sync_copy(x_hbm, x_v)
    pltpu.sync_copy(y_hbm, y_v)
    nch = PCHT // NW + jnp.where(wid < PCHT % NW, 1, 0)

    def fire_idx(k, q):
        base = (wid + k * NW) * PCH
        pltpu.async_copy(u_hbm.at[pl.ds(base, PCH)], ui[q], sui[q])
        pltpu.async_copy(v_hbm.at[pl.ds(base, PCH)], vi[q], svi[q])

    def wait_idx(q):
        pltpu.make_async_copy(u_hbm.at[pl.ds(0, PCH)], ui[q], sui[q]).wait()
        pltpu.make_async_copy(v_hbm.at[pl.ds(0, PCH)], vi[q], svi[q]).wait()

    def fire_rows(q, s):
        pltpu.async_copy(hu_hbm.at[ui[q]], ab[s], sa[s])
        pltpu.async_copy(hv_hbm.at[vi[q]], bb[s], sb[s])

    def wait_rows(s):
        pltpu.make_async_copy(hu_hbm.at[pl.ds(0, PCH)], ab[s], sa[s]).wait()
        pltpu.make_async_copy(hv_hbm.at[pl.ds(0, PCH)], bb[s], sb[s]).wait()

    def wait_g(s):
        pltpu.make_async_copy(ab[s], g_hbm.at[pl.ds(0, PCH)], sg[s]).wait()

    def wait_r(t):
        pltpu.make_async_copy(rv[t], r_hbm.at[pl.ds(0, PCH)], sr[t]).wait()

    fire_idx(0, 0)
    fire_idx(1, 1)
    wait_idx(0)
    fire_rows(0, 0)

    def blk(kk, carry):
        for j in range(12):
            k = 12 * kk + j

            @pl.when(jnp.logical_and(k >= 2, k < nch + 2))
            def _(s=(j + 1) % 3, t=j % 2):
                wait_g(s)
                wait_r(t)

            @pl.when(k + 2 < nch)
            def _(k=k, q=(j + 2) % 4):
                fire_idx(k + 2, q)

            @pl.when(k + 1 < nch)
            def _(k=k, q=(j + 1) % 4, s=(j + 1) % 3):
                wait_idx(q)
                fire_rows(q, s)

            @pl.when(k < nch)
            def _(k=k, q=j % 4, s=j % 3, t=j % 2):
                wait_rows(s)

                def grp(g, gc):
                    gsl = pl.ds(g * L, L)
                    u16 = ui[q][gsl]
                    v16 = vi[q][gsl]
                    dx = (plsc.load_gather(x_v, [u16])
                          - plsc.load_gather(x_v, [v16]))
                    dy = (plsc.load_gather(y_v, [u16])
                          - plsc.load_gather(y_v, [v16]))
                    d2 = dx * dx + dy * dy + 1e-8
                    rv[t][gsl] = d2 * _rsqrt(d2)
                    return gc

                lax.fori_loop(0, PCH // L, grp, 0)

                @plsc.parallel_loop(0, PCH, 1, unroll=4)
                def _(i):
                    for jj in range(D // L):
                        sl = pl.ds(jj * L, L)
                        ab[s][i, sl] = ab[s][i, sl] + bb[s][i, sl]

                base = (wid + k * NW) * PCH
                pltpu.async_copy(ab[s], g_hbm.at[pl.ds(base, PCH)], sg[s])
                pltpu.async_copy(rv[t], r_hbm.at[pl.ds(base, PCH)], sr[t])
        return carry

    lax.fori_loop(0, (PCHT // NW + 1 + 11) // 12, blk, 0)


def _k_head(*args):
    mesh = plsc.VectorSubcoreMesh(core_axis_name="c", subcore_axis_name="s")
    return pl.kernel(
        _head_body,
        out_type=[jax.ShapeDtypeStruct((P, D), F32),
                  jax.ShapeDtypeStruct((P,), F32)],
        mesh=mesh,
        compiler_params=pltpu.CompilerParams(needs_layout_passes=False),
        scratch_types=(
            [pltpu.VMEM((PCH,), jnp.int32)] * 8
            + [pltpu.VMEM((PCH, D), F32)] * 6
            + [pltpu.VMEM((N,), F32),
               pltpu.VMEM((N,), F32),
               pltpu.VMEM((PCH,), F32),
               pltpu.VMEM((PCH,), F32)]
            + [pltpu.SemaphoreType.DMA] * 19
        ),
    )(*args)


def _headfin_body(g_ref, r_ref, w1c_ref, w2_ref, b2_ref, o_ref):
    z = g_ref[...] + r_ref[...] * w1c_ref[...]
    o_ref[...] = lax.dot_general(
        jnp.maximum(z, 0.0), w2_ref[...], (((1,), (0,)), ((), ())),
        preferred_element_type=F32) + b2_ref[...]


def _k_headfin(g, r2, w1c2, w22, b22):
    return pl.pallas_call(
        _headfin_body,
        grid=(pl.cdiv(P, PB),),
        in_specs=[
            pl.BlockSpec((PB, D), lambda i: (i, 0)),
            pl.BlockSpec((PB, 1), lambda i: (i, 0)),
            pl.BlockSpec((1, D), lambda i: (0, 0)),
            pl.BlockSpec((D, 1), lambda i: (0, 0)),
            pl.BlockSpec((1, 1), lambda i: (0, 0)),
        ],
        out_specs=pl.BlockSpec((PB, 1), lambda i: (i, 0)),
        out_shape=jax.ShapeDtypeStruct((P, 1), F32),
    )(g, r2, w1c2, w22, b22)


def kernel(coords01, msg_edge_index, cand_pairs_uv, params):
    w1n, b1n, w2n, b2n = params["node_in"]
    e1, eb1, e2, eb2 = params["edge_head"]
    src = msg_edge_index[0]
    dst = msg_edge_index[1]
    u = cand_pairs_uv[:, 0]
    v = cand_pairs_uv[:, 1]
    x = coords01[:, 0]
    y = coords01[:, 1]
    zn = jnp.zeros((N, D), F32)
    zdeg = jnp.zeros((N, NRBF), F32)
    onescol = jnp.concatenate(
        [jnp.ones((ECH, 1), F32), jnp.zeros((ECH, NRBF - 1), F32)], axis=1)

    wa = [params["msg"][i][0][:D] for i in range(3)]
    wb = [params["msg"][i][0][D:2 * D] for i in range(3)]
    wc = [params["msg"][i][0][2 * D:] for i in range(3)]
    b1m = [params["msg"][i][1].reshape(1, D) for i in range(3)]
    w2m = [params["msg"][i][2] for i in range(3)]
    b2m = [params["msg"][i][3].reshape(1, D) for i in range(3)]
    u1a = [params["upd"][i][0][:D] for i in range(3)]
    u1b = [params["upd"][i][0][D:] for i in range(3)]
    ub1 = [params["upd"][i][1].reshape(1, D) for i in range(3)]
    u2 = [params["upd"][i][2] for i in range(3)]
    ub2 = [params["upd"][i][3].reshape(1, D) for i in range(3)]
    e1a = e1[:D]
    e1b = e1[D:2 * D]
    w1c_h = e1[2 * D].reshape(1, D)

    h, a, b = _k_node(coords01, w1n, b1n.reshape(1, D), w2n,
                      b2n.reshape(1, D), wa[0], wb[0], b1m[0])
    r, deg2 = _k_geom(x, y, src, dst, onescol, zdeg)
    c1, = _k_rbfc(r, [wc[0]])
    c2, c3 = _k_rbfc(r, [wc[1], wc[2]])
    cs = [c1, c2, c3]

    for i in range(3):
        agg2 = _k_edge(a, b, cs[i], src, dst, zn)
        if i < 2:
            nwa, nwb, nb1 = wa[i + 1], wb[i + 1], b1m[i + 1]
        else:
            nwa, nwb, nb1 = e1a, e1b, eb1.reshape(1, D)
        h, a, b = _k_post(h, agg2, deg2, w2m[i], b2m[i], u1a[i], u1b[i],
                          ub1[i], u2[i], ub2[i], nwa, nwb, nb1)

    g, rh = _k_head(a, b, x, y, u, v)
    logits = _k_headfin(g, rh.reshape(P, 1), w1c_h, e2, eb2.reshape(1, 1))
    return logits.reshape(P)


# speedup vs baseline: 1.2223x; 1.2223x over previous
import functools

import jax
import jax.numpy as jnp
import numpy as np
from jax import lax
from jax.experimental import pallas as pl
from jax.experimental.pallas import tpu as pltpu
from jax.experimental.pallas import tpu_sc as plsc

N = 10000
E = 320000
P = 500000
D = 128
NRBF = 16
NC = 2
NS = 16
NW = NC * NS
L = 16

EPW = E // NW
ECH = 80
ENCH = EPW // ECH
ECHE = 40
ENCHE = EPW // ECHE
PCH = 80
RPT = 624
PCHT = P // PCH
NB = 10
NBLK = N // NB
EB = 4096
F32 = jnp.float32


def _rsqrt(d2):
    ii = plsc.bitcast(d2, jnp.int32)
    y = plsc.bitcast(jnp.int32(0x5F3759DF) - (ii >> 1), F32)
    for _ in range(3):
        y = y * (1.5 - 0.5 * d2 * y * y)
    return y


def _dot(a, b):
    return jnp.dot(a, b, preferred_element_type=F32)


def _node_body(x_ref, w1_ref, b1_ref, w2_ref, b2_ref, wa_ref, wb_ref,
               bm_ref, h_ref, a_ref, b_ref):
    t = jnp.maximum(_dot(x_ref[...], w1_ref[...]) + b1_ref[...], 0.0)
    h = _dot(t, w2_ref[...]) + b2_ref[...]
    h_ref[...] = h
    a_ref[...] = _dot(h, wa_ref[...])
    b_ref[...] = _dot(h, wb_ref[...]) + bm_ref[...]


def _k_node(coords01, w1, b1, w2, b2, wa, wb, bm):
    full2 = pl.BlockSpec((D, D), lambda i: (0, 0))
    bias = pl.BlockSpec((1, D), lambda i: (0, 0))
    return pl.pallas_call(
        _node_body,
        grid=(NB,),
        in_specs=[
            pl.BlockSpec((NBLK, 2), lambda i: (i, 0)),
            pl.BlockSpec((2, D), lambda i: (0, 0)), bias, full2, bias,
            full2, full2, bias,
        ],
        out_specs=[pl.BlockSpec((NBLK, D), lambda i: (i, 0))] * 3,
        out_shape=[jax.ShapeDtypeStruct((N, D), F32)] * 3,
    )(coords01, w1, b1, w2, b2, wa, wb, bm)


def _make_rbfc_body(nw):
    def body(*refs):
        r_ref = refs[0]
        w_refs = refs[1:1 + nw]
        c_refs = refs[1 + nw:]
        centers = lax.broadcasted_iota(jnp.int32, (NRBF, 1), 0).astype(F32) \
            * np.float32(1.5 / (NRBF - 1))
        ft = jnp.exp(-10.0 * (r_ref[...][None, :] - centers) ** 2)
        dn = (((0,), (0,)), ((), ()))
        for w_ref, c_ref in zip(w_refs, c_refs):
            c_ref[...] = lax.dot_general(ft, w_ref[...], dn,
                                         preferred_element_type=F32)
    return body


def _k_rbfc(r, ws):
    wspec = pl.BlockSpec((NRBF, D), lambda i: (0, 0))
    return pl.pallas_call(
        _make_rbfc_body(len(ws)),
        grid=(pl.cdiv(E, EB),),
        in_specs=[pl.BlockSpec((EB,), lambda i: (i,))] + [wspec] * len(ws),
        out_specs=[pl.BlockSpec((EB, D), lambda i: (i, 0))] * len(ws),
        out_shape=[jax.ShapeDtypeStruct((E, D), F32)] * len(ws),
    )(r, *ws)


def _post_body(h_ref, aga_ref, agb_ref, dga_ref, dgb_ref, w2m_ref, b2m_ref,
               u1a_ref, u1b_ref, ub1_ref, u2_ref, ub2_ref, nwa_ref, nwb_ref,
               nb1_ref, hn_ref, a_ref, b_ref):
    h = h_ref[...]
    aggp = aga_ref[...] + agb_ref[...]
    deg = jnp.sum(dga_ref[...] + dgb_ref[...], axis=1, keepdims=True)
    agg = _dot(aggp, w2m_ref[...]) + deg * b2m_ref[...]
    t = jnp.maximum(_dot(h, u1a_ref[...]) + _dot(agg, u1b_ref[...])
                    + ub1_ref[...], 0.0)
    hn = h + _dot(t, u2_ref[...]) + ub2_ref[...]
    hn_ref[...] = hn
    a_ref[...] = _dot(hn, nwa_ref[...])
    b_ref[...] = _dot(hn, nwb_ref[...]) + nb1_ref[...]


def _k_post(h, agg2, deg2, w2m, b2m, u1a, u1b, ub1, u2, ub2, nwa, nwb, nb1):
    full2 = pl.BlockSpec((D, D), lambda i: (0, 0))
    bias = pl.BlockSpec((1, D), lambda i: (0, 0))
    nblk = pl.BlockSpec((NBLK, D), lambda i: (i, 0))
    return pl.pallas_call(
        _post_body,
        grid=(NB,),
        in_specs=[
            nblk,
            pl.BlockSpec((NBLK, D), lambda i: (i, 0)),
            pl.BlockSpec((NBLK, D), lambda i: (i + NB, 0)),
            pl.BlockSpec((NBLK, NRBF), lambda i: (i, 0)),
            pl.BlockSpec((NBLK, NRBF), lambda i: (i + NB, 0)),
            full2, bias, full2, full2, bias, full2, bias,
            full2, full2, bias,
        ],
        out_specs=[nblk] * 3,
        out_shape=[jax.ShapeDtypeStruct((N, D), F32)] * 3,
    )(h, agg2, agg2, deg2, deg2, w2m, b2m, u1a, u1b, ub1, u2, ub2,
      nwa, nwb, nb1)


def _geom_body(x_hbm, y_hbm, src_hbm, dst_hbm, ones_hbm, zdeg_hbm,
               r_hbm, deg_hbm,
               si0, si1, si2, si3, di0, di1, di2, di3,
               x_v, y_v, r_v, ones_v, deg_sp,
               ssi0, ssi1, ssi2, ssi3, sdi0, sdi1, sdi2, sdi3):
    si = [si0, si1, si2, si3]
    di = [di0, di1, di2, di3]
    ssi = [ssi0, ssi1, ssi2, ssi3]
    sdi = [sdi0, sdi1, sdi2, sdi3]
    cid = lax.axis_index("c")
    sid = lax.axis_index("s")
    wid = cid * NS + sid
    pltpu.sync_copy(x_hbm, x_v)
    pltpu.sync_copy(y_hbm, y_v)
    pltpu.sync_copy(ones_hbm, ones_v)

    @pl.when(sid == 0)
    def _():
        pltpu.sync_copy(zdeg_hbm, deg_sp)

    plsc.subcore_barrier()

    def fire_idx(k, q):
        base = wid * EPW + k * ECH
        pltpu.async_copy(src_hbm.at[pl.ds(base, ECH)], si[q], ssi[q])
        pltpu.async_copy(dst_hbm.at[pl.ds(base, ECH)], di[q], sdi[q])

    def wait_idx(q):
        pltpu.make_async_copy(src_hbm.at[pl.ds(0, ECH)], si[q], ssi[q]).wait()
        pltpu.make_async_copy(dst_hbm.at[pl.ds(0, ECH)], di[q], sdi[q]).wait()

    fire_idx(0, 0)
    fire_idx(1, 1)

    def quad(kk, carry):
        for j in range(4):
            k = 4 * kk + j

            @pl.when(k + 2 < ENCH)
            def _(k=k, q=(j + 2) % 4):
                fire_idx(k + 2, q)

            @pl.when(k < ENCH)
            def _(k=k, q=j):
                wait_idx(q)
                base = wid * EPW + k * ECH
                for jj in range(ECH // L):
                    sl = pl.ds(jj * L, L)
                    s16 = si[q][sl]
                    d16 = di[q][sl]
                    dx = (plsc.load_gather(x_v, [s16])
                          - plsc.load_gather(x_v, [d16]))
                    dy = (plsc.load_gather(y_v, [s16])
                          - plsc.load_gather(y_v, [d16]))
                    d2 = dx * dx + dy * dy + 1e-8
                    r_v[sl] = d2 * _rsqrt(d2)
                pltpu.sync_copy(r_v, r_hbm.at[pl.ds(base, ECH)])
                pltpu.sync_copy(ones_v, deg_sp.at[di[q]], add=True)
        return carry

    lax.fori_loop(0, (ENCH + 3) // 4, quad, 0)
    plsc.subcore_barrier()
    pltpu.sync_copy(deg_sp.at[pl.ds(sid * RPT, RPT)],
                    deg_hbm.at[pl.ds(cid * N + sid * RPT, RPT)])

    @pl.when(sid == 0)
    def _():
        pltpu.sync_copy(deg_sp.at[pl.ds(NS * RPT, N - NS * RPT)],
                        deg_hbm.at[pl.ds(cid * N + NS * RPT, N - NS * RPT)])


def _k_geom(*args):
    mesh = plsc.VectorSubcoreMesh(core_axis_name="c", subcore_axis_name="s")
    return pl.kernel(
        _geom_body,
        out_type=[jax.ShapeDtypeStruct((E,), F32),
                  jax.ShapeDtypeStruct((2 * N, NRBF), F32)],
        mesh=mesh,
        compiler_params=pltpu.CompilerParams(needs_layout_passes=False),
        scratch_types=(
            [pltpu.VMEM((ECH,), jnp.int32)] * 8
            + [pltpu.VMEM((N,), F32),
               pltpu.VMEM((N,), F32),
               pltpu.VMEM((ECH,), F32),
               pltpu.VMEM((ECH, NRBF), F32),
               pltpu.VMEM_SHARED((N, NRBF), F32)]
            + [pltpu.SemaphoreType.DMA] * 8
        ),
    )(*args)


def _edge_body(a_hbm, b_hbm, c_hbm, src_hbm, dst_hbm, zn_hbm, agg_hbm,
               si0, si1, si2, si3, di0, di1, di2, di3,
               a0, a1, a2, b0, b1, b2, c0, c1, c2, acc_sp,
               ssi0, ssi1, ssi2, ssi3, sdi0, sdi1, sdi2, sdi3,
               sa0, sa1, sa2, sb0, sb1, sb2, sc0, sc1, sc2,
               ss0, ss1, ss2):
    si = [si0, si1, si2, si3]
    di = [di0, di1, di2, di3]
    ab = [a0, a1, a2]
    bb = [b0, b1, b2]
    cb = [c0, c1, c2]
    ssi = [ssi0, ssi1, ssi2, ssi3]
    sdi = [sdi0, sdi1, sdi2, sdi3]
    sa = [sa0, sa1, sa2]
    sb = [sb0, sb1, sb2]
    sc = [sc0, sc1, sc2]
    ss = [ss0, ss1, ss2]
    cid = lax.axis_index("c")
    sid = lax.axis_index("s")
    wid = cid * NS + sid

    @pl.when(sid == 0)
    def _():
        pltpu.sync_copy(zn_hbm, acc_sp)

    plsc.subcore_barrier()

    def fire_idx(k, q):
        base = wid * EPW + k * ECHE
        pltpu.async_copy(src_hbm.at[pl.ds(base, ECHE)], si[q], ssi[q])
        pltpu.async_copy(dst_hbm.at[pl.ds(base, ECHE)], di[q], sdi[q])

    def wait_idx(q):
        pltpu.make_async_copy(src_hbm.at[pl.ds(0, ECHE)], si[q], ssi[q]).wait()
        pltpu.make_async_copy(dst_hbm.at[pl.ds(0, ECHE)], di[q], sdi[q]).wait()

    def fire_rows(k, q, s):
        base = wid * EPW + k * ECHE
        pltpu.async_copy(a_hbm.at[si[q]], ab[s], sa[s])
        pltpu.async_copy(b_hbm.at[di[q]], bb[s], sb[s])
        pltpu.async_copy(c_hbm.at[pl.ds(base, ECHE)], cb[s], sc[s])

    def wait_rows(s):
        pltpu.make_async_copy(a_hbm.at[pl.ds(0, ECHE)], ab[s], sa[s]).wait()
        pltpu.make_async_copy(b_hbm.at[pl.ds(0, ECHE)], bb[s], sb[s]).wait()
        pltpu.make_async_copy(c_hbm.at[pl.ds(0, ECHE)], cb[s], sc[s]).wait()

    def wait_scat(q, s):
        pltpu.make_async_copy(ab[s], acc_sp.at[di[q]], ss[s]).wait()

    fire_idx(0, 0)
    fire_idx(1, 1)
    wait_idx(0)
    fire_rows(0, 0, 0)

    def blk(kk, carry):
        for j in range(12):
            k = 12 * kk + j

            @pl.when(jnp.logical_and(k >= 2, k - 2 < ENCHE))
            def _(q=(j + 2) % 4, s=(j + 1) % 3):
                wait_scat(q, s)

            @pl.when(k + 2 < ENCHE)
            def _(k=k, q=(j + 2) % 4):
                fire_idx(k + 2, q)

            @pl.when(k + 1 < ENCHE)
            def _(k=k, q=(j + 1) % 4, s=(j + 1) % 3):
                wait_idx(q)
                fire_rows(k + 1, q, s)

            @pl.when(k < ENCHE)
            def _(k=k, q=j % 4, s=j % 3):
                wait_rows(s)

                def row_fn(i, rc):
                    for jj in range(D // L):
                        sl = pl.ds(jj * L, L)
                        z = ab[s][i, sl] + bb[s][i, sl] + cb[s][i, sl]
                        ab[s][i, sl] = jnp.maximum(z, 0.0)
                    return rc

                lax.fori_loop(0, ECHE, row_fn, 0)
                pltpu.async_copy(ab[s], acc_sp.at[di[q]], ss[s], add=True)
        return carry

    lax.fori_loop(0, (ENCHE + 11) // 12, blk, 0)
    plsc.subcore_barrier()
    pltpu.sync_copy(acc_sp.at[pl.ds(sid * RPT, RPT)],
                    agg_hbm.at[pl.ds(cid * N + sid * RPT, RPT)])

    @pl.when(sid == 0)
    def _():
        pltpu.sync_copy(acc_sp.at[pl.ds(NS * RPT, N - NS * RPT)],
                        agg_hbm.at[pl.ds(cid * N + NS * RPT, N - NS * RPT)])


def _k_edge(*args):
    mesh = plsc.VectorSubcoreMesh(core_axis_name="c", subcore_axis_name="s")
    return pl.kernel(
        _edge_body,
        out_type=jax.ShapeDtypeStruct((2 * N, D), F32),
        mesh=mesh,
        compiler_params=pltpu.CompilerParams(needs_layout_passes=False),
        scratch_types=(
            [pltpu.VMEM((ECHE,), jnp.int32)] * 8
            + [pltpu.VMEM((ECHE, D), F32)] * 9
            + [pltpu.VMEM_SHARED((N, D), F32)]
            + [pltpu.SemaphoreType.DMA] * 20
        ),
    )(*args)


def _head_body(hu_hbm, hv_hbm, x_hbm, y_hbm, u_hbm, v_hbm, w1c_hbm, w2_hbm,
               b2_hbm, out_hbm,
               ui0, ui1, ui2, ui3, vi0, vi1, vi2, vi3,
               a0, a1, b0, b1,
               x_v, y_v, w1c_v, w2_v, b2_v, r_v, m_v, o0, o1,
               sui0, sui1, sui2, sui3, svi0, svi1, svi2, svi3,
               sa0, sa1, sb0, sb1, so0, so1):
    ui = [ui0, ui1, ui2, ui3]
    vi = [vi0, vi1, vi2, vi3]
    ab = [a0, a1]
    bb = [b0, b1]
    ov = [o0, o1]
    sui = [sui0, sui1, sui2, sui3]
    svi = [svi0, svi1, svi2, svi3]
    sa = [sa0, sa1]
    sb = [sb0, sb1]
    so = [so0, so1]
    cid = lax.axis_index("c")
    sid = lax.axis_index("s")
    wid = cid * NS + sid
    pltpu.sync_copy(x_hbm, x_v)
    pltpu.sync_copy(y_hbm, y_v)
    pltpu.sync_copy(w1c_hbm, w1c_v)
    pltpu.sync_copy(w2_hbm, w2_v)
    pltpu.sync_copy(b2_hbm, b2_v)
    nch = PCHT // NW + jnp.where(wid < PCHT % NW, 1, 0)
    iota = lax.iota(jnp.int32, L)

    def fire_idx(k, q):
        base = (wid + k * NW) * PCH
        pltpu.async_copy(u_hbm.at[pl.ds(base, PCH)], ui[q], sui[q])
        pltpu.async_copy(v_hbm.at[pl.ds(base, PCH)], vi[q], svi[q])

    def wait_idx(q):
        pltpu.make_async_copy(u_hbm.at[pl.ds(0, PCH)], ui[q], sui[q]).wait()
        pltpu.make_async_copy(v_hbm.at[pl.ds(0, PCH)], vi[q], svi[q]).wait()

    def fire_rows(q, s):
        pltpu.async_copy(hu_hbm.at[ui[q]], ab[s], sa[s])
        pltpu.async_copy(hv_hbm.at[vi[q]], bb[s], sb[s])

    def wait_rows(s):
        pltpu.make_async_copy(hu_hbm.at[pl.ds(0, PCH)], ab[s], sa[s]).wait()
        pltpu.make_async_copy(hv_hbm.at[pl.ds(0, PCH)], bb[s], sb[s]).wait()

    fire_idx(0, 0)
    fire_idx(1, 1)
    wait_idx(0)
    fire_rows(0, 0)

    def quad(kk, carry):
        for j in range(4):
            k = 4 * kk + j

            @pl.when(k + 2 < nch)
            def _(k=k, q=(j + 2) % 4):
                fire_idx(k + 2, q)

            @pl.when(k + 1 < nch)
            def _(k=k, q=(j + 1) % 4, s=(j + 1) % 2):
                wait_idx(q)
                fire_rows(q, s)

            @pl.when(k < nch)
            def _(k=k, q=j, s=j % 2):
                wait_rows(s)

                @pl.when(k >= 2)
                def _(s=s):
                    pltpu.make_async_copy(
                        ov[s], out_hbm.at[pl.ds(0, PCH)], so[s]).wait()

                w1cs = [w1c_v[pl.ds(jj * L, L)] for jj in range(D // L)]
                w2s = [w2_v[pl.ds(jj * L, L)] for jj in range(D // L)]

                def grp(g, gc):
                    gsl = pl.ds(g * L, L)
                    u16 = ui[q][gsl]
                    v16 = vi[q][gsl]
                    dx = (plsc.load_gather(x_v, [u16])
                          - plsc.load_gather(x_v, [v16]))
                    dy = (plsc.load_gather(y_v, [u16])
                          - plsc.load_gather(y_v, [v16]))
                    d2 = dx * dx + dy * dy + 1e-8
                    r_v[...] = d2 * _rsqrt(d2)
                    for p in range(L):
                        rp = plsc.load_gather(
                            r_v, [jnp.full((L,), p, jnp.int32)])
                        row = g * L + p
                        ts = []
                        for jj in range(D // L):
                            sl = pl.ds(jj * L, L)
                            z = (ab[s][row, sl] + bb[s][row, sl]
                                 + rp * w1cs[jj])
                            ts.append(jnp.maximum(z, 0.0) * w2s[jj])
                        while len(ts) > 1:
                            ts = [ts[i] + ts[i + 1]
                                  for i in range(0, len(ts), 2)]
                        m_v[p] = ts[0]
                    t = b2_v[...]
                    for jj in range(L):
                        t = t + plsc.load_gather(
                            m_v, [iota, jnp.full((L,), jj, jnp.int32)])
                    ov[s][gsl] = t
                    return gc

                lax.fori_loop(0, PCH // L, grp, 0)
                base = (wid + k * NW) * PCH
                pltpu.async_copy(ov[s], out_hbm.at[pl.ds(base, PCH)], so[s])
        return carry

    lax.fori_loop(0, (PCHT // NW + 1 + 3) // 4, quad, 0)
    pltpu.make_async_copy(ov[0], out_hbm.at[pl.ds(0, PCH)], so[0]).wait()
    pltpu.make_async_copy(ov[1], out_hbm.at[pl.ds(0, PCH)], so[1]).wait()


def _k_head(*args):
    mesh = plsc.VectorSubcoreMesh(core_axis_name="c", subcore_axis_name="s")
    return pl.kernel(
        _head_body,
        out_type=jax.ShapeDtypeStruct((P,), F32),
        mesh=mesh,
        compiler_params=pltpu.CompilerParams(needs_layout_passes=False),
        scratch_types=(
            [pltpu.VMEM((PCH,), jnp.int32)] * 8
            + [pltpu.VMEM((PCH, D), F32)] * 4
            + [pltpu.VMEM((N,), F32),
               pltpu.VMEM((N,), F32),
               pltpu.VMEM((D,), F32),
               pltpu.VMEM((D,), F32),
               pltpu.VMEM((L,), F32),
               pltpu.VMEM((L,), F32),
               pltpu.VMEM((L, L), F32),
               pltpu.VMEM((PCH,), F32),
               pltpu.VMEM((PCH,), F32)]
            + [pltpu.SemaphoreType.DMA] * 14
        ),
    )(*args)


def kernel(coords01, msg_edge_index, cand_pairs_uv, params):
    w1n, b1n, w2n, b2n = params["node_in"]
    e1, eb1, e2, eb2 = params["edge_head"]
    src = msg_edge_index[0]
    dst = msg_edge_index[1]
    u = cand_pairs_uv[:, 0]
    v = cand_pairs_uv[:, 1]
    x = coords01[:, 0]
    y = coords01[:, 1]
    zn = jnp.zeros((N, D), F32)
    zdeg = jnp.zeros((N, NRBF), F32)
    onescol = jnp.concatenate(
        [jnp.ones((ECH, 1), F32), jnp.zeros((ECH, NRBF - 1), F32)], axis=1)

    wa = [params["msg"][i][0][:D] for i in range(3)]
    wb = [params["msg"][i][0][D:2 * D] for i in range(3)]
    wc = [params["msg"][i][0][2 * D:] for i in range(3)]
    b1m = [params["msg"][i][1].reshape(1, D) for i in range(3)]
    w2m = [params["msg"][i][2] for i in range(3)]
    b2m = [params["msg"][i][3].reshape(1, D) for i in range(3)]
    u1a = [params["upd"][i][0][:D] for i in range(3)]
    u1b = [params["upd"][i][0][D:] for i in range(3)]
    ub1 = [params["upd"][i][1].reshape(1, D) for i in range(3)]
    u2 = [params["upd"][i][2] for i in range(3)]
    ub2 = [params["upd"][i][3].reshape(1, D) for i in range(3)]
    e1a = e1[:D]
    e1b = e1[D:2 * D]
    w1c_h = e1[2 * D]
    w2_h = e2[:, 0]
    b2_h = jnp.full((L,), eb2[0], F32)

    h, a, b = _k_node(coords01, w1n, b1n.reshape(1, D), w2n,
                      b2n.reshape(1, D), wa[0], wb[0], b1m[0])
    r, deg2 = _k_geom(x, y, src, dst, onescol, zdeg)
    c1, = _k_rbfc(r, [wc[0]])
    c2, c3 = _k_rbfc(r, [wc[1], wc[2]])
    cs = [c1, c2, c3]

    for i in range(3):
        agg2 = _k_edge(a, b, cs[i], src, dst, zn)
        if i < 2:
            nwa, nwb, nb1 = wa[i + 1], wb[i + 1], b1m[i + 1]
        else:
            nwa, nwb, nb1 = e1a, e1b, eb1.reshape(1, D)
        h, a, b = _k_post(h, agg2, deg2, w2m[i], b2m[i], u1a[i], u1b[i],
                          ub1[i], u2[i], ub2[i], nwa, nwb, nb1)

    return _k_head(a, b, x, y, u, v, w1c_h, w2_h, b2_h)
